# baseline (speedup 1.0000x reference)
import functools

import jax
import jax.numpy as jnp
from jax import lax
from jax.experimental import pallas as pl
from jax.experimental.pallas import tpu as pltpu
from jax.experimental.pallas import tpu_sc as plsc

N_NODES = 10000
N_EDGES = 320000
D_IN = 128
D_HID = 64
D_OUT = 128

NC = 2
NS = 16
NW = NC * NS

N_PAD = 10240
ROWS_PER_SUB = N_PAD // NS
CHUNK = 512
E_PAD = 327680
CPT = E_PAD // NW // CHUNK
BLK = 256
GRID = N_PAD // BLK

_mesh = plsc.VectorSubcoreMesh(
    core_axis_name="c", subcore_axis_name="s", num_cores=NC, num_subcores=NS
)
_sc_params = pltpu.CompilerParams(use_tc_tiling_on_sc=False)


def _sc_agg_body(u_hbm, zero_hbm, src_hbm, dst_hbm, out_hbm,
                 acc_sh, sidx0, didx0, sidx1, didx1, buf0, buf1,
                 gs0, gs1, ss0, ss1):
    c = lax.axis_index("c")
    s = lax.axis_index("s")
    wid = c * NS + s
    row0 = s * ROWS_PER_SUB
    pltpu.sync_copy(zero_hbm.at[pl.ds(row0, ROWS_PER_SUB)],
                    acc_sh.at[pl.ds(row0, ROWS_PER_SUB)])
    plsc.subcore_barrier()

    def body(i, carry):
        j0 = 2 * i
        pltpu.sync_copy(src_hbm.at[wid, j0], sidx0)
        pltpu.sync_copy(dst_hbm.at[wid, j0], didx0)
        g0 = pltpu.async_copy(u_hbm.at[sidx0], buf0, gs0)
        pltpu.sync_copy(src_hbm.at[wid, j0 + 1], sidx1)
        pltpu.sync_copy(dst_hbm.at[wid, j0 + 1], didx1)
        g1 = pltpu.async_copy(u_hbm.at[sidx1], buf1, gs1)
        g0.wait()
        s0 = pltpu.async_copy(buf0, acc_sh.at[didx0], ss0, add=True)
        g1.wait()
        s1 = pltpu.async_copy(buf1, acc_sh.at[didx1], ss1, add=True)
        s0.wait()
        s1.wait()
        return carry

    lax.fori_loop(0, CPT // 2, body, 0)
    plsc.subcore_barrier()
    pltpu.sync_copy(acc_sh.at[pl.ds(row0, ROWS_PER_SUB)],
                    out_hbm.at[c, pl.ds(row0, ROWS_PER_SUB)])


_sc_agg = functools.partial(
    pl.kernel,
    out_type=jax.ShapeDtypeStruct((NC, N_PAD, D_HID), jnp.float32),
    mesh=_mesh,
    scratch_types=[
        pltpu.VMEM_SHARED((N_PAD, D_HID), jnp.float32),
        pltpu.VMEM((CHUNK,), jnp.int32),
        pltpu.VMEM((CHUNK,), jnp.int32),
        pltpu.VMEM((CHUNK,), jnp.int32),
        pltpu.VMEM((CHUNK,), jnp.int32),
        pltpu.VMEM((CHUNK, D_HID), jnp.float32),
        pltpu.VMEM((CHUNK, D_HID), jnp.float32),
        pltpu.SemaphoreType.DMA,
        pltpu.SemaphoreType.DMA,
        pltpu.SemaphoreType.DMA,
        pltpu.SemaphoreType.DMA,
    ],
    compiler_params=_sc_params,
)(_sc_agg_body)


def _sc_deg_body(ones_hbm, zero_hbm, dst_hbm, out_hbm,
                 deg_sh, dst_v, ones_v):
    c = lax.axis_index("c")
    s = lax.axis_index("s")
    wid = c * NS + s
    row0 = s * ROWS_PER_SUB
    pltpu.sync_copy(zero_hbm.at[pl.ds(row0, ROWS_PER_SUB)],
                    deg_sh.at[pl.ds(row0, ROWS_PER_SUB)])
    pltpu.sync_copy(ones_hbm, ones_v)
    plsc.subcore_barrier()

    def body(j, carry):
        pltpu.sync_copy(dst_hbm.at[wid, j], dst_v)
        pltpu.sync_copy(ones_v, deg_sh.at[dst_v], add=True)
        return carry

    lax.fori_loop(0, CPT, body, 0)
    plsc.subcore_barrier()
    pltpu.sync_copy(deg_sh.at[pl.ds(row0, ROWS_PER_SUB)],
                    out_hbm.at[c, pl.ds(row0, ROWS_PER_SUB)])


_sc_deg = functools.partial(
    pl.kernel,
    out_type=jax.ShapeDtypeStruct((NC, N_PAD, 16), jnp.float32),
    mesh=_mesh,
    scratch_types=[
        pltpu.VMEM_SHARED((N_PAD, 16), jnp.float32),
        pltpu.VMEM((CHUNK,), jnp.int32),
        pltpu.VMEM((CHUNK, 16), jnp.float32),
    ],
    compiler_params=_sc_params,
)(_sc_deg_body)


def _tc_pre0_body(x_ref, degA_ref, degB_ref, w_ref, u_ref, dis_ref):
    deg = degA_ref[:, :1] + degB_ref[:, :1] + 1.0
    dis = jnp.where(deg > 0, lax.rsqrt(deg), 0.0)
    h = jnp.dot(x_ref[...], w_ref[...], preferred_element_type=jnp.float32)
    u_ref[...] = dis * h
    dis_ref[...] = dis


def _tc_pre0(x, degA, degB, W0):
    return pl.pallas_call(
        _tc_pre0_body,
        grid=(GRID,),
        in_specs=[
            pl.BlockSpec((BLK, D_IN), lambda i: (i, 0)),
            pl.BlockSpec((BLK, 16), lambda i: (i, 0)),
            pl.BlockSpec((BLK, 16), lambda i: (i, 0)),
            pl.BlockSpec((D_IN, D_HID), lambda i: (0, 0)),
        ],
        out_specs=[
            pl.BlockSpec((BLK, D_HID), lambda i: (i, 0)),
            pl.BlockSpec((BLK, 1), lambda i: (i, 0)),
        ],
        out_shape=[
            jax.ShapeDtypeStruct((N_PAD, D_HID), jnp.float32),
            jax.ShapeDtypeStruct((N_PAD, 1), jnp.float32),
        ],
    )(x, degA, degB, W0)


def _ln_relu(t, g, be, eps=1e-5):
    mu = jnp.mean(t, axis=-1, keepdims=True)
    var = jnp.mean((t - mu) ** 2, axis=-1, keepdims=True)
    tn = (t - mu) * lax.rsqrt(var + eps) * g + be
    return jnp.maximum(tn, 0.0)


def _tc_mid_body(has_res, aggA_ref, aggB_ref, u_ref, dis_ref, b_ref, g_ref,
                 be_ref, wn_ref, *rest):
    if has_res:
        hprev_ref, h_ref, un_ref = rest
    else:
        h_ref, un_ref = rest
    dis = dis_ref[...]
    t = dis * (aggA_ref[...] + aggB_ref[...] + u_ref[...]) + b_ref[...]
    h = _ln_relu(t, g_ref[...], be_ref[...])
    if has_res:
        h = hprev_ref[...] + h
    h_ref[...] = h
    un_ref[...] = dis * jnp.dot(h, wn_ref[...],
                                preferred_element_type=jnp.float32)


def _tc_mid(aggA, aggB, u, dis, b, g, be, Wn, hprev=None):
    has_res = hprev is not None
    ins = [aggA, aggB, u, dis, b.reshape(1, D_HID), g.reshape(1, D_HID),
           be.reshape(1, D_HID), Wn]
    in_specs = [
        pl.BlockSpec((BLK, D_HID), lambda i: (i, 0)),
        pl.BlockSpec((BLK, D_HID), lambda i: (i, 0)),
        pl.BlockSpec((BLK, D_HID), lambda i: (i, 0)),
        pl.BlockSpec((BLK, 1), lambda i: (i, 0)),
        pl.BlockSpec((1, D_HID), lambda i: (0, 0)),
        pl.BlockSpec((1, D_HID), lambda i: (0, 0)),
        pl.BlockSpec((1, D_HID), lambda i: (0, 0)),
        pl.BlockSpec((D_HID, D_HID), lambda i: (0, 0)),
    ]
    if has_res:
        ins.append(hprev)
        in_specs.append(pl.BlockSpec((BLK, D_HID), lambda i: (i, 0)))
    return pl.pallas_call(
        functools.partial(_tc_mid_body, has_res),
        grid=(GRID,),
        in_specs=in_specs,
        out_specs=[
            pl.BlockSpec((BLK, D_HID), lambda i: (i, 0)),
            pl.BlockSpec((BLK, D_HID), lambda i: (i, 0)),
        ],
        out_shape=[
            jax.ShapeDtypeStruct((N_PAD, D_HID), jnp.float32),
            jax.ShapeDtypeStruct((N_PAD, D_HID), jnp.float32),
        ],
    )(*ins)


def _tc_fin_body(aggA_ref, aggB_ref, u_ref, dis_ref, b_ref, g_ref, be_ref,
                 hprev_ref, wa_ref, ba_ref, wb_ref, bb_ref, out_ref, acc_ref):
    i = pl.program_id(0)
    dis = dis_ref[...]
    t = dis * (aggA_ref[...] + aggB_ref[...] + u_ref[...]) + b_ref[...]
    h = hprev_ref[...] + _ln_relu(t, g_ref[...], be_ref[...])
    row = i * BLK + lax.broadcasted_iota(jnp.int32, (BLK, 1), 0)
    h = jnp.where(row < N_NODES, h, 0.0)
    part = jnp.sum(h, axis=0, keepdims=True)

    @pl.when(i == 0)
    def _():
        acc_ref[...] = jnp.zeros_like(acc_ref)

    acc_ref[...] += part

    @pl.when(i == GRID - 1)
    def _():
        mean = acc_ref[...] * (1.0 / N_NODES)
        hid = jnp.maximum(
            jnp.dot(mean, wa_ref[...], preferred_element_type=jnp.float32)
            + ba_ref[...], 0.0)
        out_ref[...] = (
            jnp.dot(hid, wb_ref[...], preferred_element_type=jnp.float32)
            + bb_ref[...])


def _tc_fin(aggA, aggB, u, dis, b, g, be, hprev, Wa, ba, Wb, bb):
    return pl.pallas_call(
        _tc_fin_body,
        grid=(GRID,),
        in_specs=[
            pl.BlockSpec((BLK, D_HID), lambda i: (i, 0)),
            pl.BlockSpec((BLK, D_HID), lambda i: (i, 0)),
            pl.BlockSpec((BLK, D_HID), lambda i: (i, 0)),
            pl.BlockSpec((BLK, 1), lambda i: (i, 0)),
            pl.BlockSpec((1, D_HID), lambda i: (0, 0)),
            pl.BlockSpec((1, D_HID), lambda i: (0, 0)),
            pl.BlockSpec((1, D_HID), lambda i: (0, 0)),
            pl.BlockSpec((BLK, D_HID), lambda i: (i, 0)),
            pl.BlockSpec((D_HID, D_HID), lambda i: (0, 0)),
            pl.BlockSpec((1, D_HID), lambda i: (0, 0)),
            pl.BlockSpec((D_HID, D_OUT), lambda i: (0, 0)),
            pl.BlockSpec((1, D_OUT), lambda i: (0, 0)),
        ],
        out_specs=pl.BlockSpec((1, D_OUT), lambda i: (0, 0)),
        out_shape=jax.ShapeDtypeStruct((1, D_OUT), jnp.float32),
        scratch_shapes=[pltpu.VMEM((1, D_HID), jnp.float32)],
    )(aggA, aggB, u, dis, b.reshape(1, D_HID), g.reshape(1, D_HID),
      be.reshape(1, D_HID), hprev, Wa, ba.reshape(1, D_HID), Wb,
      bb.reshape(1, D_OUT))


def kernel(x, edge_index, W0, b0, W1, b1, W2, b2, g0, be0, g1, be1, g2, be2,
           Wa, ba, Wb, bb):
    src = edge_index[0].astype(jnp.int32)
    dst = edge_index[1].astype(jnp.int32)
    pad = jnp.full((E_PAD - N_EDGES,), N_NODES, dtype=jnp.int32)
    src_r = jnp.concatenate([src, pad]).reshape(NW, CPT, CHUNK)
    dst_r = jnp.concatenate([dst, pad]).reshape(NW, CPT, CHUNK)

    x_pad = jnp.zeros((N_PAD, D_IN), jnp.float32).at[:N_NODES].set(x)
    zeros64 = jnp.zeros((N_PAD, D_HID), jnp.float32)
    zeros16 = jnp.zeros((N_PAD, 16), jnp.float32)
    ones16 = jnp.ones((CHUNK, 16), jnp.float32)

    degp = _sc_deg(ones16, zeros16, dst_r)
    u0, dis = _tc_pre0(x_pad, degp[0], degp[1], W0)

    agg0 = _sc_agg(u0, zeros64, src_r, dst_r)
    h1, u1 = _tc_mid(agg0[0], agg0[1], u0, dis, b0, g0, be0, W1)

    agg1 = _sc_agg(u1, zeros64, src_r, dst_r)
    h2, u2 = _tc_mid(agg1[0], agg1[1], u1, dis, b1, g1, be1, W2, hprev=h1)

    agg2 = _sc_agg(u2, zeros64, src_r, dst_r)
    return _tc_fin(agg2[0], agg2[1], u2, dis, b2, g2, be2, h2, Wa, ba, Wb, bb)


# speedup vs baseline: 2.5279x; 2.5279x over previous
import functools

import jax
import jax.numpy as jnp
from jax import lax
from jax.experimental import pallas as pl
from jax.experimental.pallas import tpu as pltpu
from jax.experimental.pallas import tpu_sc as plsc

N_NODES = 10000
N_EDGES = 320000
D_IN = 128
D_HID = 64
D_OUT = 128

NC = 2
NS = 16
NW = NC * NS

N_PAD = 10240
ROWS_PER_SUB = N_PAD // NS
CHUNK = 512
E_PAD = 327680
CPT = E_PAD // NW // CHUNK
BLK = 256
GRID = N_PAD // BLK

_mesh = plsc.VectorSubcoreMesh(
    core_axis_name="c", subcore_axis_name="s", num_cores=NC, num_subcores=NS
)
_sc_params = pltpu.CompilerParams(use_tc_tiling_on_sc=False)


def _sc_agg_body(u_hbm, zero_hbm, src_hbm, dst_hbm, out_hbm,
                 u_sh, acc_sh, sidx_v, didx_v, buf_v):
    c = lax.axis_index("c")
    s = lax.axis_index("s")
    wid = c * NS + s
    row0 = s * ROWS_PER_SUB
    pltpu.sync_copy(u_hbm.at[pl.ds(row0, ROWS_PER_SUB)],
                    u_sh.at[pl.ds(row0, ROWS_PER_SUB)])
    pltpu.sync_copy(zero_hbm.at[pl.ds(row0, ROWS_PER_SUB)],
                    acc_sh.at[pl.ds(row0, ROWS_PER_SUB)])
    plsc.subcore_barrier()

    def body(j, carry):
        pltpu.sync_copy(src_hbm.at[wid, j], sidx_v)
        pltpu.sync_copy(dst_hbm.at[wid, j], didx_v)
        pltpu.sync_copy(u_sh.at[sidx_v], buf_v)
        pltpu.sync_copy(buf_v, acc_sh.at[didx_v], add=True)
        return carry

    lax.fori_loop(0, CPT, body, 0)
    plsc.subcore_barrier()
    pltpu.sync_copy(acc_sh.at[pl.ds(row0, ROWS_PER_SUB)],
                    out_hbm.at[c, pl.ds(row0, ROWS_PER_SUB)])


_sc_agg = functools.partial(
    pl.kernel,
    out_type=jax.ShapeDtypeStruct((NC, N_PAD, D_HID), jnp.float32),
    mesh=_mesh,
    scratch_types=[
        pltpu.VMEM_SHARED((N_PAD, D_HID), jnp.float32),
        pltpu.VMEM_SHARED((N_PAD, D_HID), jnp.float32),
        pltpu.VMEM((CHUNK,), jnp.int32),
        pltpu.VMEM((CHUNK,), jnp.int32),
        pltpu.VMEM((CHUNK, D_HID), jnp.float32),
    ],
    compiler_params=_sc_params,
)(_sc_agg_body)


def _sc_deg_body(ones_hbm, zero_hbm, dst_hbm, out_hbm,
                 deg_sh, dst_v, ones_v):
    c = lax.axis_index("c")
    s = lax.axis_index("s")
    wid = c * NS + s
    row0 = s * ROWS_PER_SUB
    pltpu.sync_copy(zero_hbm.at[pl.ds(row0, ROWS_PER_SUB)],
                    deg_sh.at[pl.ds(row0, ROWS_PER_SUB)])
    pltpu.sync_copy(ones_hbm, ones_v)
    plsc.subcore_barrier()

    def body(j, carry):
        pltpu.sync_copy(dst_hbm.at[wid, j], dst_v)
        pltpu.sync_copy(ones_v, deg_sh.at[dst_v], add=True)
        return carry

    lax.fori_loop(0, CPT, body, 0)
    plsc.subcore_barrier()
    pltpu.sync_copy(deg_sh.at[pl.ds(row0, ROWS_PER_SUB)],
                    out_hbm.at[c, pl.ds(row0, ROWS_PER_SUB)])


_sc_deg = functools.partial(
    pl.kernel,
    out_type=jax.ShapeDtypeStruct((NC, N_PAD, 16), jnp.float32),
    mesh=_mesh,
    scratch_types=[
        pltpu.VMEM_SHARED((N_PAD, 16), jnp.float32),
        pltpu.VMEM((CHUNK,), jnp.int32),
        pltpu.VMEM((CHUNK, 16), jnp.float32),
    ],
    compiler_params=_sc_params,
)(_sc_deg_body)


def _tc_pre0_body(x_ref, degp_ref, w_ref, u_ref, dis_ref):
    deg = degp_ref[0, :, :1] + degp_ref[1, :, :1] + 1.0
    dis = jnp.where(deg > 0, lax.rsqrt(deg), 0.0)
    h = jnp.dot(x_ref[...], w_ref[...], preferred_element_type=jnp.float32)
    u_ref[...] = dis * h
    dis_ref[...] = dis


def _tc_pre0(x, degp, W0):
    return pl.pallas_call(
        _tc_pre0_body,
        out_shape=[
            jax.ShapeDtypeStruct((N_PAD, D_HID), jnp.float32),
            jax.ShapeDtypeStruct((N_PAD, 1), jnp.float32),
        ],
    )(x, degp, W0)


def _ln_relu(t, g, be, eps=1e-5):
    mu = jnp.mean(t, axis=-1, keepdims=True)
    var = jnp.mean((t - mu) ** 2, axis=-1, keepdims=True)
    tn = (t - mu) * lax.rsqrt(var + eps) * g + be
    return jnp.maximum(tn, 0.0)


def _tc_mid_body(has_res, aggp_ref, u_ref, dis_ref, b_ref, g_ref,
                 be_ref, wn_ref, *rest):
    if has_res:
        hprev_ref, h_ref, un_ref = rest
    else:
        h_ref, un_ref = rest
    dis = dis_ref[...]
    t = dis * (aggp_ref[0] + aggp_ref[1] + u_ref[...]) + b_ref[...]
    h = _ln_relu(t, g_ref[...], be_ref[...])
    if has_res:
        h = hprev_ref[...] + h
    h_ref[...] = h
    un_ref[...] = dis * jnp.dot(h, wn_ref[...],
                                preferred_element_type=jnp.float32)


def _tc_mid(aggp, u, dis, b, g, be, Wn, hprev=None):
    has_res = hprev is not None
    ins = [aggp, u, dis, b.reshape(1, D_HID), g.reshape(1, D_HID),
           be.reshape(1, D_HID), Wn]
    if has_res:
        ins.append(hprev)
    return pl.pallas_call(
        functools.partial(_tc_mid_body, has_res),
        out_shape=[
            jax.ShapeDtypeStruct((N_PAD, D_HID), jnp.float32),
            jax.ShapeDtypeStruct((N_PAD, D_HID), jnp.float32),
        ],
    )(*ins)


def _tc_fin_body(aggp_ref, u_ref, dis_ref, b_ref, g_ref, be_ref,
                 hprev_ref, wa_ref, ba_ref, wb_ref, bb_ref, out_ref):
    dis = dis_ref[...]
    t = dis * (aggp_ref[0] + aggp_ref[1] + u_ref[...]) + b_ref[...]
    h = hprev_ref[...] + _ln_relu(t, g_ref[...], be_ref[...])
    row = lax.broadcasted_iota(jnp.int32, (N_PAD, 1), 0)
    h = jnp.where(row < N_NODES, h, 0.0)
    mean = jnp.sum(h, axis=0, keepdims=True) * (1.0 / N_NODES)
    hid = jnp.maximum(
        jnp.dot(mean, wa_ref[...], preferred_element_type=jnp.float32)
        + ba_ref[...], 0.0)
    out_ref[...] = (
        jnp.dot(hid, wb_ref[...], preferred_element_type=jnp.float32)
        + bb_ref[...])


def _tc_fin(aggp, u, dis, b, g, be, hprev, Wa, ba, Wb, bb):
    return pl.pallas_call(
        _tc_fin_body,
        out_shape=jax.ShapeDtypeStruct((1, D_OUT), jnp.float32),
    )(aggp, u, dis, b.reshape(1, D_HID), g.reshape(1, D_HID),
      be.reshape(1, D_HID), hprev, Wa, ba.reshape(1, D_HID), Wb,
      bb.reshape(1, D_OUT))


def kernel(x, edge_index, W0, b0, W1, b1, W2, b2, g0, be0, g1, be1, g2, be2,
           Wa, ba, Wb, bb):
    src = edge_index[0].astype(jnp.int32)
    dst = edge_index[1].astype(jnp.int32)
    pad = jnp.full((E_PAD - N_EDGES,), N_NODES, dtype=jnp.int32)
    src_r = jnp.concatenate([src, pad]).reshape(NW, CPT, CHUNK)
    dst_r = jnp.concatenate([dst, pad]).reshape(NW, CPT, CHUNK)

    x_pad = jnp.zeros((N_PAD, D_IN), jnp.float32).at[:N_NODES].set(x)
    zeros64 = jnp.zeros((N_PAD, D_HID), jnp.float32)
    zeros16 = jnp.zeros((N_PAD, 16), jnp.float32)
    ones16 = jnp.ones((CHUNK, 16), jnp.float32)

    degp = _sc_deg(ones16, zeros16, dst_r)
    u0, dis = _tc_pre0(x_pad, degp, W0)

    agg0 = _sc_agg(u0, zeros64, src_r, dst_r)
    h1, u1 = _tc_mid(agg0, u0, dis, b0, g0, be0, W1)

    agg1 = _sc_agg(u1, zeros64, src_r, dst_r)
    h2, u2 = _tc_mid(agg1, u1, dis, b1, g1, be1, W2, hprev=h1)

    agg2 = _sc_agg(u2, zeros64, src_r, dst_r)
    return _tc_fin(agg2, u2, dis, b2, g2, be2, h2, Wa, ba, Wb, bb)
